# baseline (speedup 1.0000x reference)
import jax
import jax.numpy as jnp
from jax import lax
from jax.experimental import pallas as pl
from jax.experimental.pallas import tpu as pltpu
from jax.experimental.pallas import tpu_sc as plsc

N = 32768
C = 2048
NW = 32
RPW = N // NW
L = 16
G = C // L
B = 4
NBUF = 6
NBAT = RPW // B
NMAIN = (NBAT // NBUF) * NBUF
NCH = 8
NT = C // (NCH * L)


def _body(x_hbm, m_hbm, o_hbm, mvm, *bufs):
    xbufs = bufs[0:NBUF]
    obufs = bufs[NBUF:2 * NBUF]
    sins = bufs[2 * NBUF:3 * NBUF]
    souts = bufs[3 * NBUF:4 * NBUF]
    wid = lax.axis_index("s") * 2 + lax.axis_index("c")
    base = wid * RPW
    pltpu.sync_copy(m_hbm.at[pl.ds(base, RPW)], mvm.at[pl.ds(0, RPW)])
    lanes = lax.iota(jnp.int32, L)

    def start_in(b, k):
        rs = base + b * B
        pltpu.async_copy(x_hbm.at[pl.ds(rs, B)], xbufs[k], sins[k])

    def wait_in(k):
        pltpu.make_async_copy(x_hbm.at[pl.ds(0, B)], xbufs[k], sins[k]).wait()

    def start_out(b, k):
        rs = base + b * B
        pltpu.async_copy(obufs[k], o_hbm.at[pl.ds(rs, B)], souts[k])

    def wait_out(k):
        pltpu.make_async_copy(obufs[k], o_hbm.at[pl.ds(0, B)], souts[k]).wait()

    def do_batch(b, k):
        xbk = xbufs[k]
        obk = obufs[k]
        mvec = mvm[pl.ds(b * B, L)]
        for rb in range(B):
            mm = mvec[rb]
            rvec = lanes * 0 + rb

            def gath(vs):
                return tuple(plsc.load_gather(xbk, [rvec, v]) for v in vs)

            def adv(vs):
                return tuple((v + NCH * L) & (C - 1) for v in vs)

            vs = tuple(
                (lanes + (mm + o * L)) & (C - 1) for o in range(NCH))
            vals = gath(vs)
            vs = adv(vs)

            def tstep(t, carry):
                vs, vals = carry
                col = t * (NCH * L)
                for o in range(NCH):
                    obk[rb, pl.ds(col + o * L, L)] = vals[o]
                return adv(vs), gath(vs)

            lax.fori_loop(0, NT, tstep, (vs, vals), unroll=2)

    for j in range(NBUF - 1):
        start_in(j, j)

    def outer(bb, carry):
        for k in range(NBUF):
            b = NBUF * bb + k
            nb = b + NBUF - 1
            nk = (k + NBUF - 1) % NBUF

            @pl.when(nb < NBAT)
            def _():
                start_in(nb, nk)

            wait_in(k)

            @pl.when(b >= NBUF)
            def _():
                wait_out(k)

            do_batch(b, k)
            start_out(b, k)
        return carry

    lax.fori_loop(0, NMAIN // NBUF, outer, 0)
    for b in range(NMAIN, NBAT):
        k = b % NBUF
        wait_in(k)
        wait_out(k)
        do_batch(b, k)
        start_out(b, k)
    for k in range(NBUF):
        wait_out(k)


def kernel(x, map):
    m = map.reshape(-1).astype(jnp.int32)
    mesh = plsc.VectorSubcoreMesh(core_axis_name="c", subcore_axis_name="s")
    f = pl.kernel(
        _body,
        out_type=jax.ShapeDtypeStruct((N, C), jnp.float32),
        mesh=mesh,
        scratch_types=(
            [pltpu.VMEM((RPW + L,), jnp.int32)]
            + [pltpu.VMEM((B, C), jnp.float32) for _ in range(2 * NBUF)]
            + [pltpu.SemaphoreType.DMA for _ in range(2 * NBUF)]
        ),
        compiler_params=pltpu.CompilerParams(needs_layout_passes=False),
    )
    return f(x, m)


# speedup vs baseline: 1.0368x; 1.0368x over previous
import jax
import jax.numpy as jnp
from jax import lax
from jax.experimental import pallas as pl
from jax.experimental.pallas import tpu as pltpu
from jax.experimental.pallas import tpu_sc as plsc

N = 32768
C = 2048
NW = 32
RPW = N // NW
L = 16
G = C // L
B = 4
NBUF = 5
NBAT = RPW // B
NMAIN = (NBAT // NBUF) * NBUF
NCH = 8
NT = C // (NCH * L)


def _body(x_hbm, m_hbm, o_hbm, mvm, *bufs):
    xbufs = bufs[0:NBUF]
    obufs = bufs[NBUF:2 * NBUF]
    sins = bufs[2 * NBUF:3 * NBUF]
    souts = bufs[3 * NBUF:4 * NBUF]
    wid = lax.axis_index("s") * 2 + lax.axis_index("c")
    base = wid * RPW
    pltpu.sync_copy(m_hbm.at[pl.ds(base, RPW)], mvm.at[pl.ds(0, RPW)])
    lanes = lax.iota(jnp.int32, L)

    def start_in(b, k):
        rs = base + b * B
        pltpu.async_copy(x_hbm.at[pl.ds(rs, B)], xbufs[k], sins[k])

    def wait_in(k):
        pltpu.make_async_copy(x_hbm.at[pl.ds(0, B)], xbufs[k], sins[k]).wait()

    def start_out(b, k):
        rs = base + b * B
        pltpu.async_copy(obufs[k], o_hbm.at[pl.ds(rs, B)], souts[k])

    def wait_out(k):
        pltpu.make_async_copy(obufs[k], o_hbm.at[pl.ds(0, B)], souts[k]).wait()

    def do_batch(b, k):
        xbk = xbufs[k]
        obk = obufs[k]
        mvec = mvm[pl.ds(b * B, L)]
        for rb in range(B):
            mm = mvec[rb]
            rvec = lanes * 0 + rb

            def gath(vs):
                return tuple(plsc.load_gather(xbk, [rvec, v]) for v in vs)

            def adv(vs):
                return tuple((v + NCH * L) & (C - 1) for v in vs)

            vs = tuple(
                (lanes + (mm + o * L)) & (C - 1) for o in range(NCH))
            vals = gath(vs)
            vs = adv(vs)

            def tstep(t, carry):
                vs, vals = carry
                col = t * (NCH * L)
                for o in range(NCH):
                    obk[rb, pl.ds(col + o * L, L)] = vals[o]
                return adv(vs), gath(vs)

            lax.fori_loop(0, NT, tstep, (vs, vals), unroll=2)

    for j in range(NBUF - 1):
        start_in(j, j)

    def outer(bb, carry):
        for k in range(NBUF):
            b = NBUF * bb + k
            nb = b + NBUF - 1
            nk = (k + NBUF - 1) % NBUF

            @pl.when(nb < NBAT)
            def _():
                start_in(nb, nk)

            wait_in(k)

            @pl.when(b >= NBUF)
            def _():
                wait_out(k)

            do_batch(b, k)
            start_out(b, k)
        return carry

    lax.fori_loop(0, NMAIN // NBUF, outer, 0)
    for b in range(NMAIN, NBAT):
        k = b % NBUF
        wait_in(k)
        wait_out(k)
        do_batch(b, k)
        start_out(b, k)
    for k in range(NBUF):
        wait_out(k)


def kernel(x, map):
    m = map.reshape(-1).astype(jnp.int32)
    mesh = plsc.VectorSubcoreMesh(core_axis_name="c", subcore_axis_name="s")
    f = pl.kernel(
        _body,
        out_type=jax.ShapeDtypeStruct((N, C), jnp.float32),
        mesh=mesh,
        scratch_types=(
            [pltpu.VMEM((RPW + L,), jnp.int32)]
            + [pltpu.VMEM((B, C), jnp.float32) for _ in range(2 * NBUF)]
            + [pltpu.SemaphoreType.DMA for _ in range(2 * NBUF)]
        ),
        compiler_params=pltpu.CompilerParams(needs_layout_passes=False),
    )
    return f(x, m)


# speedup vs baseline: 1.1211x; 1.0813x over previous
import jax
import jax.numpy as jnp
from jax import lax
from jax.experimental import pallas as pl
from jax.experimental.pallas import tpu as pltpu
from jax.experimental.pallas import tpu_sc as plsc

N = 32768
C = 2048
NW = 32
RPW = N // NW
L = 16
G = C // L
B = 4
NBUF = 4
NBAT = RPW // B
NMAIN = (NBAT // NBUF) * NBUF
NCH = 8
NT = C // (NCH * L)


def _body(x_hbm, m_hbm, o_hbm, mvm, *bufs):
    xbufs = bufs[0:NBUF]
    obufs = bufs[NBUF:2 * NBUF]
    sins = bufs[2 * NBUF:3 * NBUF]
    souts = bufs[3 * NBUF:4 * NBUF]
    wid = lax.axis_index("s") * 2 + lax.axis_index("c")
    base = wid * RPW
    pltpu.sync_copy(m_hbm.at[pl.ds(base, RPW)], mvm.at[pl.ds(0, RPW)])
    lanes = lax.iota(jnp.int32, L)

    def start_in(b, k):
        rs = base + b * B
        pltpu.async_copy(x_hbm.at[pl.ds(rs, B)], xbufs[k], sins[k])

    def wait_in(k):
        pltpu.make_async_copy(x_hbm.at[pl.ds(0, B)], xbufs[k], sins[k]).wait()

    def start_out(b, k):
        rs = base + b * B
        pltpu.async_copy(obufs[k], o_hbm.at[pl.ds(rs, B)], souts[k])

    def wait_out(k):
        pltpu.make_async_copy(obufs[k], o_hbm.at[pl.ds(0, B)], souts[k]).wait()

    def do_batch(b, k):
        xbk = xbufs[k]
        obk = obufs[k]
        mvec = mvm[pl.ds(b * B, L)]
        for rb in range(B):
            mm = mvec[rb]
            rvec = lanes * 0 + rb

            def gath(vs):
                return tuple(plsc.load_gather(xbk, [rvec, v]) for v in vs)

            def adv(vs):
                return tuple((v + NCH * L) & (C - 1) for v in vs)

            vs = tuple(
                (lanes + (mm + o * L)) & (C - 1) for o in range(NCH))
            vals = gath(vs)
            vs = adv(vs)

            def tstep(t, carry):
                vs, vals = carry
                col = t * (NCH * L)
                for o in range(NCH):
                    obk[rb, pl.ds(col + o * L, L)] = vals[o]
                return adv(vs), gath(vs)

            lax.fori_loop(0, NT, tstep, (vs, vals), unroll=2)

    for j in range(NBUF - 1):
        start_in(j, j)

    def outer(bb, carry):
        for k in range(NBUF):
            b = NBUF * bb + k
            nb = b + NBUF - 1
            nk = (k + NBUF - 1) % NBUF

            @pl.when(nb < NBAT)
            def _():
                start_in(nb, nk)

            wait_in(k)

            @pl.when(b >= NBUF)
            def _():
                wait_out(k)

            do_batch(b, k)
            start_out(b, k)
        return carry

    lax.fori_loop(0, NMAIN // NBUF, outer, 0)
    for b in range(NMAIN, NBAT):
        k = b % NBUF
        wait_in(k)
        wait_out(k)
        do_batch(b, k)
        start_out(b, k)
    for k in range(NBUF):
        wait_out(k)


def kernel(x, map):
    m = map.reshape(-1).astype(jnp.int32)
    mesh = plsc.VectorSubcoreMesh(core_axis_name="c", subcore_axis_name="s")
    f = pl.kernel(
        _body,
        out_type=jax.ShapeDtypeStruct((N, C), jnp.float32),
        mesh=mesh,
        scratch_types=(
            [pltpu.VMEM((RPW + L,), jnp.int32)]
            + [pltpu.VMEM((B, C), jnp.float32) for _ in range(2 * NBUF)]
            + [pltpu.SemaphoreType.DMA for _ in range(2 * NBUF)]
        ),
        compiler_params=pltpu.CompilerParams(needs_layout_passes=False),
    )
    return f(x, m)
